# baseline (speedup 1.0000x reference)
import functools

import jax
import jax.numpy as jnp
from jax import lax
from jax.experimental import pallas as pl
from jax.experimental.pallas import tpu as pltpu
from jax.experimental.pallas import tpu_sc as plsc

_EPS = 1e-5


def _build_gather_call(B, S, V, D):
    N = B * S
    info = plsc.get_sparse_core_info()
    NC, NS = info.num_cores, info.num_subcores
    NW = NC * NS
    R = N // NW
    K = 32
    G = R // K
    WPB = S // R
    assert N % NW == 0 and R % K == 0 and S % R == 0

    mesh = plsc.VectorSubcoreMesh(core_axis_name="c", subcore_axis_name="s")

    @functools.partial(
        pl.kernel,
        mesh=mesh,
        out_type=jax.ShapeDtypeStruct((N, D), jnp.float32),
        compiler_params=pltpu.CompilerParams(needs_layout_passes=False),
        scratch_types=[
            pltpu.VMEM((R,), jnp.int32),
            pltpu.VMEM((2, K, D), jnp.float32),
            pltpu.SemaphoreType.DMA((2,)),
            pltpu.SemaphoreType.DMA((2,)),
        ],
    )
    def gather_kernel(ids_hbm, word_hbm, out_hbm, ids_v, rows_v, gsem, osem):
        wid = lax.axis_index("s") * NC + lax.axis_index("c")
        base = wid * R
        pltpu.sync_copy(
            ids_hbm.at[wid // WPB, pl.ds((wid % WPB) * R, R)], ids_v)

        def start_gather(g, b):
            pltpu.async_copy(word_hbm.at[ids_v.at[pl.ds(g * K, K)]],
                             rows_v.at[b], gsem.at[b])

        start_gather(0, 0)

        def chunk_body(g, carry):
            b = lax.rem(g, 2)
            nb = 1 - b

            @pl.when(g >= 1)
            def _():
                pltpu.make_async_copy(
                    rows_v.at[nb], out_hbm.at[pl.ds(base, K)], osem.at[nb]
                ).wait()

            @pl.when(g + 1 < G)
            def _():
                start_gather(g + 1, nb)

            pltpu.make_async_copy(word_hbm.at[ids_v.at[pl.ds(g * K, K)]],
                                  rows_v.at[b], gsem.at[b]).wait()
            pltpu.async_copy(rows_v.at[b],
                             out_hbm.at[pl.ds(base + g * K, K)], osem.at[b])
            return carry

        lax.fori_loop(0, G, chunk_body, 0)
        pltpu.make_async_copy(
            rows_v.at[(G - 1) % 2], out_hbm.at[pl.ds(base, K)],
            osem.at[(G - 1) % 2]
        ).wait()

    return gather_kernel


def _ln_block_kernel(g_ref, p_ref, gamma_ref, beta_ref, o_ref):
    x = g_ref[...] + p_ref[...]
    d = x.shape[1]
    mean = jnp.sum(x, axis=1, keepdims=True) * (1.0 / d)
    ex2 = jnp.sum(x * x, axis=1, keepdims=True) * (1.0 / d)
    rstd = lax.rsqrt(ex2 - mean * mean + _EPS)
    o_ref[...] = ((x - mean) * rstd) * gamma_ref[...] + beta_ref[...]


def _ln_chain_kernel(prev_ref, g_ref, p_ref, gamma_ref, beta_ref, o_ref):
    del prev_ref
    _ln_block_kernel(g_ref, p_ref, gamma_ref, beta_ref, o_ref)


def _build_ln_call(N, S, D, BLK, B, SS, seq_off, chained):
    pb_slice = SS // BLK
    s_blocks = S // BLK
    pb_off = seq_off // BLK
    grid = (pb_slice, B)

    data_specs = [
        pl.BlockSpec((BLK, D), lambda i, j: (j * pb_slice + i, 0)),
        pl.BlockSpec((BLK, D), lambda i, j: (pb_off + i, 0)),
        pl.BlockSpec((1, D), lambda i, j: (0, 0)),
        pl.BlockSpec((1, D), lambda i, j: (0, 0)),
    ]
    if chained:
        in_specs = [pl.BlockSpec(memory_space=pl.ANY)] + data_specs
        body = _ln_chain_kernel
        aliases = {0: 0}
    else:
        in_specs = data_specs
        body = _ln_block_kernel
        aliases = {}

    return pl.pallas_call(
        body,
        grid=grid,
        in_specs=in_specs,
        out_specs=pl.BlockSpec(
            (BLK, D),
            lambda i, j: (j * s_blocks + pb_off + i, 0),
        ),
        out_shape=jax.ShapeDtypeStruct((N, D), jnp.float32),
        input_output_aliases=aliases,
        compiler_params=pltpu.CompilerParams(
            dimension_semantics=("arbitrary", "arbitrary"),
        ),
    )


def kernel(input_ids, word_table, pos_table, ln_gamma, ln_beta):
    B, S = input_ids.shape
    V, D = word_table.shape
    N = B * S
    NSLICES = 4
    SS = S // NSLICES
    BLK = min(SS, 1024)
    ids = input_ids.astype(jnp.int32)
    gamma2 = ln_gamma.reshape(1, D)
    beta2 = ln_beta.reshape(1, D)

    gather = _build_gather_call(B, SS, V, D)
    gathered = [gather(ids[:, s * SS:(s + 1) * SS], word_table)
                for s in range(NSLICES)]

    out = None
    for s in range(NSLICES):
        ln = _build_ln_call(N, S, D, BLK, B, SS, s * SS, chained=s > 0)
        if s == 0:
            out = ln(gathered[s], pos_table, gamma2, beta2)
        else:
            out = ln(out, gathered[s], pos_table, gamma2, beta2)
    return out.reshape(B, S, D)


# speedup vs baseline: 1.0649x; 1.0649x over previous
import functools

import jax
import jax.numpy as jnp
from jax import lax
from jax.experimental import pallas as pl
from jax.experimental.pallas import tpu as pltpu
from jax.experimental.pallas import tpu_sc as plsc

_EPS = 1e-5


def _build_gather_call(B, S, V, D):
    N = B * S
    info = plsc.get_sparse_core_info()
    NC, NS = info.num_cores, info.num_subcores
    NW = NC * NS
    R = N // NW
    K = 32
    G = R // K
    WPB = S // R
    assert N % NW == 0 and R % K == 0 and S % R == 0

    mesh = plsc.VectorSubcoreMesh(core_axis_name="c", subcore_axis_name="s")

    @functools.partial(
        pl.kernel,
        mesh=mesh,
        out_type=jax.ShapeDtypeStruct((N, D), jnp.float32),
        compiler_params=pltpu.CompilerParams(needs_layout_passes=False),
        scratch_types=[
            pltpu.VMEM((R,), jnp.int32),
            pltpu.VMEM((2, K, D), jnp.float32),
            pltpu.SemaphoreType.DMA((2,)),
            pltpu.SemaphoreType.DMA((2,)),
        ],
    )
    def gather_kernel(ids_hbm, word_hbm, out_hbm, ids_v, rows_v, gsem, osem):
        wid = lax.axis_index("s") * NC + lax.axis_index("c")
        base = wid * R
        pltpu.sync_copy(
            ids_hbm.at[wid // WPB, pl.ds((wid % WPB) * R, R)], ids_v)

        def start_gather(g, b):
            pltpu.async_copy(word_hbm.at[ids_v.at[pl.ds(g * K, K)]],
                             rows_v.at[b], gsem.at[b])

        start_gather(0, 0)

        def chunk_body(g, carry):
            b = lax.rem(g, 2)
            nb = 1 - b

            @pl.when(g >= 1)
            def _():
                pltpu.make_async_copy(
                    rows_v.at[nb], out_hbm.at[pl.ds(base, K)], osem.at[nb]
                ).wait()

            @pl.when(g + 1 < G)
            def _():
                start_gather(g + 1, nb)

            pltpu.make_async_copy(word_hbm.at[ids_v.at[pl.ds(g * K, K)]],
                                  rows_v.at[b], gsem.at[b]).wait()
            pltpu.async_copy(rows_v.at[b],
                             out_hbm.at[pl.ds(base + g * K, K)], osem.at[b])
            return carry

        lax.fori_loop(0, G, chunk_body, 0)
        pltpu.make_async_copy(
            rows_v.at[(G - 1) % 2], out_hbm.at[pl.ds(base, K)],
            osem.at[(G - 1) % 2]
        ).wait()

    return gather_kernel


def _ln_block_kernel(g_ref, p_ref, gamma_ref, beta_ref, o_ref):
    x = g_ref[...] + p_ref[...]
    d = x.shape[1]
    mean = jnp.sum(x, axis=1, keepdims=True) * (1.0 / d)
    ex2 = jnp.sum(x * x, axis=1, keepdims=True) * (1.0 / d)
    rstd = lax.rsqrt(ex2 - mean * mean + _EPS)
    o_ref[...] = ((x - mean) * rstd) * gamma_ref[...] + beta_ref[...]


def _ln_chain_kernel(prev_ref, g_ref, p_ref, gamma_ref, beta_ref, o_ref):
    del prev_ref
    _ln_block_kernel(g_ref, p_ref, gamma_ref, beta_ref, o_ref)


def _build_ln_call(N, S, D, BLK, B, SS, seq_off, chained):
    pb_slice = SS // BLK
    s_blocks = S // BLK
    pb_off = seq_off // BLK
    grid = (pb_slice, B)

    data_specs = [
        pl.BlockSpec((BLK, D), lambda i, j: (j * pb_slice + i, 0)),
        pl.BlockSpec((BLK, D), lambda i, j: (pb_off + i, 0)),
        pl.BlockSpec((1, D), lambda i, j: (0, 0)),
        pl.BlockSpec((1, D), lambda i, j: (0, 0)),
    ]
    if chained:
        in_specs = [pl.BlockSpec(memory_space=pl.ANY)] + data_specs
        body = _ln_chain_kernel
        aliases = {0: 0}
    else:
        in_specs = data_specs
        body = _ln_block_kernel
        aliases = {}

    return pl.pallas_call(
        body,
        grid=grid,
        in_specs=in_specs,
        out_specs=pl.BlockSpec(
            (BLK, D),
            lambda i, j: (j * s_blocks + pb_off + i, 0),
        ),
        out_shape=jax.ShapeDtypeStruct((N, D), jnp.float32),
        input_output_aliases=aliases,
        compiler_params=pltpu.CompilerParams(
            dimension_semantics=("arbitrary", "arbitrary"),
        ),
    )


def kernel(input_ids, word_table, pos_table, ln_gamma, ln_beta):
    B, S = input_ids.shape
    V, D = word_table.shape
    N = B * S
    NSLICES = 2
    SS = S // NSLICES
    BLK = min(SS, 1024)
    ids = input_ids.astype(jnp.int32)
    gamma2 = ln_gamma.reshape(1, D)
    beta2 = ln_beta.reshape(1, D)

    gather = _build_gather_call(B, SS, V, D)
    gathered = [gather(ids[:, s * SS:(s + 1) * SS], word_table)
                for s in range(NSLICES)]

    out = None
    for s in range(NSLICES):
        ln = _build_ln_call(N, S, D, BLK, B, SS, s * SS, chained=s > 0)
        if s == 0:
            out = ln(gathered[s], pos_table, gamma2, beta2)
        else:
            out = ln(out, gathered[s], pos_table, gamma2, beta2)
    return out.reshape(B, S, D)


# speedup vs baseline: 1.0714x; 1.0061x over previous
import functools

import jax
import jax.numpy as jnp
from jax import lax
from jax.experimental import pallas as pl
from jax.experimental.pallas import tpu as pltpu
from jax.experimental.pallas import tpu_sc as plsc

_EPS = 1e-5


def _build_gather_call(B, S, V, D):
    N = B * S
    info = plsc.get_sparse_core_info()
    NC, NS = info.num_cores, info.num_subcores
    NW = NC * NS
    R = N // NW
    K = 16
    NB = 4
    G = R // K
    WPB = S // R
    assert N % NW == 0 and R % K == 0 and S % R == 0 and G >= NB

    mesh = plsc.VectorSubcoreMesh(core_axis_name="c", subcore_axis_name="s")

    @functools.partial(
        pl.kernel,
        mesh=mesh,
        out_type=jax.ShapeDtypeStruct((N, D), jnp.float32),
        compiler_params=pltpu.CompilerParams(needs_layout_passes=False),
        scratch_types=[
            pltpu.VMEM((R,), jnp.int32),
            pltpu.VMEM((NB, K, D), jnp.float32),
            pltpu.SemaphoreType.DMA((NB,)),
            pltpu.SemaphoreType.DMA((NB,)),
        ],
    )
    def gather_kernel(ids_hbm, word_hbm, out_hbm, ids_v, rows_v, gsem, osem):
        wid = lax.axis_index("s") * NC + lax.axis_index("c")
        base = wid * R
        pltpu.sync_copy(
            ids_hbm.at[wid // WPB, pl.ds((wid % WPB) * R, R)], ids_v)

        def start_gather(g, b):
            pltpu.async_copy(word_hbm.at[ids_v.at[pl.ds(g * K, K)]],
                             rows_v.at[b], gsem.at[b])

        def wait_writeback(b):
            pltpu.make_async_copy(
                rows_v.at[b], out_hbm.at[pl.ds(base, K)], osem.at[b]
            ).wait()

        for g in range(NB - 1):
            start_gather(g, g)

        def chunk_body(g, carry):
            b = lax.rem(g, NB)
            pltpu.make_async_copy(word_hbm.at[ids_v.at[pl.ds(g * K, K)]],
                                  rows_v.at[b], gsem.at[b]).wait()
            pltpu.async_copy(rows_v.at[b],
                             out_hbm.at[pl.ds(base + g * K, K)], osem.at[b])

            @pl.when(g + NB - 1 < G)
            def _():
                nxt = lax.rem(g + NB - 1, NB)

                @pl.when(g >= 1)
                def _():
                    wait_writeback(nxt)

                start_gather(g + NB - 1, nxt)

            return carry

        lax.fori_loop(0, G, chunk_body, 0)
        for k in range(NB):
            wait_writeback((G - NB + k) % NB)

    return gather_kernel


def _ln_block_kernel(g_ref, p_ref, gamma_ref, beta_ref, o_ref):
    x = g_ref[...] + p_ref[...]
    d = x.shape[1]
    mean = jnp.sum(x, axis=1, keepdims=True) * (1.0 / d)
    ex2 = jnp.sum(x * x, axis=1, keepdims=True) * (1.0 / d)
    rstd = lax.rsqrt(ex2 - mean * mean + _EPS)
    o_ref[...] = ((x - mean) * rstd) * gamma_ref[...] + beta_ref[...]


def _ln_chain_kernel(prev_ref, g_ref, p_ref, gamma_ref, beta_ref, o_ref):
    del prev_ref
    _ln_block_kernel(g_ref, p_ref, gamma_ref, beta_ref, o_ref)


def _build_ln_call(N, S, D, BLK, B, SS, seq_off, chained):
    pb_slice = SS // BLK
    s_blocks = S // BLK
    pb_off = seq_off // BLK
    grid = (pb_slice, B)

    data_specs = [
        pl.BlockSpec((BLK, D), lambda i, j: (j * pb_slice + i, 0)),
        pl.BlockSpec((BLK, D), lambda i, j: (pb_off + i, 0)),
        pl.BlockSpec((1, D), lambda i, j: (0, 0)),
        pl.BlockSpec((1, D), lambda i, j: (0, 0)),
    ]
    if chained:
        in_specs = [pl.BlockSpec(memory_space=pl.ANY)] + data_specs
        body = _ln_chain_kernel
        aliases = {0: 0}
    else:
        in_specs = data_specs
        body = _ln_block_kernel
        aliases = {}

    return pl.pallas_call(
        body,
        grid=grid,
        in_specs=in_specs,
        out_specs=pl.BlockSpec(
            (BLK, D),
            lambda i, j: (j * s_blocks + pb_off + i, 0),
        ),
        out_shape=jax.ShapeDtypeStruct((N, D), jnp.float32),
        input_output_aliases=aliases,
        compiler_params=pltpu.CompilerParams(
            dimension_semantics=("arbitrary", "arbitrary"),
        ),
    )


def kernel(input_ids, word_table, pos_table, ln_gamma, ln_beta):
    B, S = input_ids.shape
    V, D = word_table.shape
    N = B * S
    NSLICES = 2
    SS = S // NSLICES
    BLK = min(SS, 1024)
    ids = input_ids.astype(jnp.int32)
    gamma2 = ln_gamma.reshape(1, D)
    beta2 = ln_beta.reshape(1, D)

    gather = _build_gather_call(B, SS, V, D)
    gathered = [gather(ids[:, s * SS:(s + 1) * SS], word_table)
                for s in range(NSLICES)]

    out = None
    for s in range(NSLICES):
        ln = _build_ln_call(N, S, D, BLK, B, SS, s * SS, chained=s > 0)
        if s == 0:
            out = ln(gathered[s], pos_table, gamma2, beta2)
        else:
            out = ln(out, gathered[s], pos_table, gamma2, beta2)
    return out.reshape(B, S, D)


# speedup vs baseline: 1.0744x; 1.0028x over previous
import functools

import jax
import jax.numpy as jnp
from jax import lax
from jax.experimental import pallas as pl
from jax.experimental.pallas import tpu as pltpu
from jax.experimental.pallas import tpu_sc as plsc

_EPS = 1e-5


def _build_gather_call(B, S, V, D, soff):
    N = B * S
    info = plsc.get_sparse_core_info()
    NC, NS = info.num_cores, info.num_subcores
    NW = NC * NS
    R = N // NW
    K = 16
    NB = 4
    G = R // K
    WPB = S // R
    assert N % NW == 0 and R % K == 0 and S % R == 0 and G >= NB

    mesh = plsc.VectorSubcoreMesh(core_axis_name="c", subcore_axis_name="s")

    @functools.partial(
        pl.kernel,
        mesh=mesh,
        out_type=jax.ShapeDtypeStruct((N, D), jnp.float32),
        compiler_params=pltpu.CompilerParams(needs_layout_passes=False),
        scratch_types=[
            pltpu.VMEM((R,), jnp.int32),
            pltpu.VMEM((NB, K, D), jnp.float32),
            pltpu.SemaphoreType.DMA((NB,)),
            pltpu.SemaphoreType.DMA((NB,)),
        ],
    )
    def gather_kernel(ids_hbm, word_hbm, out_hbm, ids_v, rows_v, gsem, osem):
        wid = lax.axis_index("s") * NC + lax.axis_index("c")
        base = wid * R
        pltpu.sync_copy(
            ids_hbm.at[wid // WPB, pl.ds(soff + (wid % WPB) * R, R)], ids_v)

        def start_gather(g, b):
            pltpu.async_copy(word_hbm.at[ids_v.at[pl.ds(g * K, K)]],
                             rows_v.at[b], gsem.at[b])

        def wait_writeback(b):
            pltpu.make_async_copy(
                rows_v.at[b], out_hbm.at[pl.ds(base, K)], osem.at[b]
            ).wait()

        for g in range(NB - 1):
            start_gather(g, g)

        def chunk_body(g, carry):
            b = lax.rem(g, NB)
            pltpu.make_async_copy(word_hbm.at[ids_v.at[pl.ds(g * K, K)]],
                                  rows_v.at[b], gsem.at[b]).wait()
            pltpu.async_copy(rows_v.at[b],
                             out_hbm.at[pl.ds(base + g * K, K)], osem.at[b])

            @pl.when(g + NB - 1 < G)
            def _():
                nxt = lax.rem(g + NB - 1, NB)

                @pl.when(g >= 1)
                def _():
                    wait_writeback(nxt)

                start_gather(g + NB - 1, nxt)

            return carry

        lax.fori_loop(0, G, chunk_body, 0)
        for k in range(NB):
            wait_writeback((G - NB + k) % NB)

    return gather_kernel


def _ln_block_kernel(g_ref, p_ref, gamma_ref, beta_ref, o_ref):
    x = g_ref[...] + p_ref[...]
    d = x.shape[1]
    mean = jnp.sum(x, axis=1, keepdims=True) * (1.0 / d)
    ex2 = jnp.sum(x * x, axis=1, keepdims=True) * (1.0 / d)
    rstd = lax.rsqrt(ex2 - mean * mean + _EPS)
    o_ref[...] = ((x - mean) * rstd) * gamma_ref[...] + beta_ref[...]


def _ln_chain_kernel(prev_ref, g_ref, p_ref, gamma_ref, beta_ref, o_ref):
    del prev_ref
    _ln_block_kernel(g_ref, p_ref, gamma_ref, beta_ref, o_ref)


def _build_ln_call(N, S, D, BLK, B, SS, seq_off, chained):
    pb_slice = SS // BLK
    s_blocks = S // BLK
    pb_off = seq_off // BLK
    grid = (pb_slice, B)

    data_specs = [
        pl.BlockSpec((BLK, D), lambda i, j: (j * pb_slice + i, 0)),
        pl.BlockSpec((BLK, D), lambda i, j: (pb_off + i, 0)),
        pl.BlockSpec((1, D), lambda i, j: (0, 0)),
        pl.BlockSpec((1, D), lambda i, j: (0, 0)),
    ]
    if chained:
        in_specs = [pl.BlockSpec(memory_space=pl.ANY)] + data_specs
        body = _ln_chain_kernel
        aliases = {0: 0}
    else:
        in_specs = data_specs
        body = _ln_block_kernel
        aliases = {}

    return pl.pallas_call(
        body,
        grid=grid,
        in_specs=in_specs,
        out_specs=pl.BlockSpec(
            (BLK, D),
            lambda i, j: (j * s_blocks + pb_off + i, 0),
        ),
        out_shape=jax.ShapeDtypeStruct((N, D), jnp.float32),
        input_output_aliases=aliases,
        compiler_params=pltpu.CompilerParams(
            dimension_semantics=("arbitrary", "arbitrary"),
        ),
    )


def kernel(input_ids, word_table, pos_table, ln_gamma, ln_beta):
    B, S = input_ids.shape
    V, D = word_table.shape
    N = B * S
    NSLICES = 2
    SS = S // NSLICES
    BLK = min(SS, 1024)
    ids = input_ids.astype(jnp.int32)
    gamma2 = ln_gamma.reshape(1, D)
    beta2 = ln_beta.reshape(1, D)

    gathered = [_build_gather_call(B, SS, V, D, s * SS)(ids, word_table)
                for s in range(NSLICES)]

    out = None
    for s in range(NSLICES):
        ln = _build_ln_call(N, S, D, BLK, B, SS, s * SS, chained=s > 0)
        if s == 0:
            out = ln(gathered[s], pos_table, gamma2, beta2)
        else:
            out = ln(out, gathered[s], pos_table, gamma2, beta2)
    return out.reshape(B, S, D)
